# baseline (speedup 1.0000x reference)
import jax
import jax.numpy as jnp
from jax import lax
from jax.experimental import pallas as pl
from jax.experimental.pallas import tpu as pltpu
from jax.experimental.pallas import tpu_sc as plsc

N = 10000
E = 320000
D = 128
H = 128
C = 10
B = 128

NC, NS = 2, 16
NW = NC * NS
N_PAD = 10240
BLK = 2000
NBLK = N // BLK
K = 128
NB = 80
NPH = 4
NB_PH = NB // NPH
NPAIR = NB_PH // 2
E_PW = NB * K
E_PAD = NW * E_PW
E_PW_DEG = E // NW
ROWS_PER_SUB = N_PAD // NS

_mesh = plsc.VectorSubcoreMesh(
    core_axis_name="c", subcore_axis_name="s", num_cores=NC, num_subcores=NS)


def _deg_body(ei_hbm, out_hbm, dstv, hist):
    c = lax.axis_index("c")
    s = lax.axis_index("s")
    wid = s * NC + c
    zeros16 = jnp.zeros((16,), jnp.float32)

    def _zero(i, carry):
        hist[pl.ds(i * 16, 16)] = zeros16
        return carry

    lax.fori_loop(0, N // 16, _zero, 0)
    pltpu.sync_copy(ei_hbm.at[pl.ds(E + wid * E_PW_DEG, E_PW_DEG)], dstv)
    ones16 = jnp.ones((16,), jnp.float32)

    def _acc(i, carry):
        idx = dstv[pl.ds(i * 16, 16)]
        plsc.addupdate_scatter(hist, [idx], ones16)
        return carry

    lax.fori_loop(0, E_PW_DEG // 16, _acc, 0)
    pltpu.sync_copy(hist, out_hbm.at[wid])


_deg_kernel = pl.kernel(
    _deg_body,
    out_type=jax.ShapeDtypeStruct((NW, N), jnp.float32),
    mesh=_mesh,
    scratch_types=[
        pltpu.VMEM((E_PW_DEG,), jnp.int32),
        pltpu.VMEM((N,), jnp.float32),
    ],
    compiler_params=pltpu.CompilerParams(needs_layout_passes=False),
)


def _msg_body(hs_hbm, src_hbm, dst_hbm, zero_hbm, out_hbm,
              srcv, dstv, rows_a, rows_b, acc_sh,
              gsem_a, gsem_a2, gsem_b, gsem_b2,
              ssem_a, ssem_a2, ssem_b, ssem_b2):
    c = lax.axis_index("c")
    s = lax.axis_index("s")
    wid = s * NC + c
    pltpu.sync_copy(zero_hbm, acc_sh.at[pl.ds(s * ROWS_PER_SUB, ROWS_PER_SUB)])
    plsc.subcore_barrier()

    KH = K // 2

    def _gather(j, rows, sem_lo, sem_hi):
        pltpu.async_copy(hs_hbm.at[srcv.at[j, 0]],
                         rows.at[pl.ds(0, KH)], sem_lo)
        pltpu.async_copy(hs_hbm.at[srcv.at[j, 1]],
                         rows.at[pl.ds(KH, KH)], sem_hi)

    def _gwait(rows, sem_lo, sem_hi):
        pltpu.make_async_copy(hs_hbm.at[srcv.at[0, 0]],
                              rows.at[pl.ds(0, KH)], sem_lo).wait()
        pltpu.make_async_copy(hs_hbm.at[srcv.at[0, 0]],
                              rows.at[pl.ds(KH, KH)], sem_hi).wait()

    def _scatter(j, rows, sem_lo, sem_hi):
        pltpu.async_copy(rows.at[pl.ds(0, KH)],
                         acc_sh.at[dstv.at[j, 0]], sem_lo, add=True)
        pltpu.async_copy(rows.at[pl.ds(KH, KH)],
                         acc_sh.at[dstv.at[j, 1]], sem_hi, add=True)

    def _swait(rows, sem_lo, sem_hi):
        pltpu.make_async_copy(rows.at[pl.ds(0, KH)],
                              acc_sh.at[dstv.at[0, 0]], sem_lo).wait()
        pltpu.make_async_copy(rows.at[pl.ds(KH, KH)],
                              acc_sh.at[dstv.at[0, 0]], sem_hi).wait()

    for ph in range(NPH):
        pltpu.sync_copy(src_hbm.at[wid, ph], srcv)
        pltpu.sync_copy(dst_hbm.at[wid, ph], dstv)
        _gather(0, rows_a, gsem_a, gsem_a2)

        def _pair(p, carry):
            j = 2 * p
            _gwait(rows_a, gsem_a, gsem_a2)

            @pl.when(p > 0)
            def _b_free():
                _swait(rows_b, ssem_b, ssem_b2)

            _gather(j + 1, rows_b, gsem_b, gsem_b2)
            _scatter(j, rows_a, ssem_a, ssem_a2)
            _gwait(rows_b, gsem_b, gsem_b2)

            @pl.when(p < NPAIR - 1)
            def _a_free():
                _swait(rows_a, ssem_a, ssem_a2)
                _gather(j + 2, rows_a, gsem_a, gsem_a2)

            _scatter(j + 1, rows_b, ssem_b, ssem_b2)
            return carry

        lax.fori_loop(0, NPAIR, _pair, 0)
        _swait(rows_a, ssem_a, ssem_a2)
        _swait(rows_b, ssem_b, ssem_b2)
    plsc.subcore_barrier()
    pltpu.sync_copy(acc_sh.at[pl.ds(s * ROWS_PER_SUB, ROWS_PER_SUB)],
                    out_hbm.at[c, pl.ds(s * ROWS_PER_SUB, ROWS_PER_SUB)])


_msg_kernel = pl.kernel(
    _msg_body,
    out_type=jax.ShapeDtypeStruct((NC, N_PAD, H), jnp.float32),
    mesh=_mesh,
    scratch_types=[
        pltpu.VMEM((NB_PH, 2, K // 2), jnp.int32),
        pltpu.VMEM((NB_PH, 2, K // 2), jnp.int32),
        pltpu.VMEM((K, H), jnp.float32),
        pltpu.VMEM((K, H), jnp.float32),
        pltpu.VMEM_SHARED((N_PAD, H), jnp.float32),
        pltpu.SemaphoreType.DMA,
        pltpu.SemaphoreType.DMA,
        pltpu.SemaphoreType.DMA,
        pltpu.SemaphoreType.DMA,
        pltpu.SemaphoreType.DMA,
        pltpu.SemaphoreType.DMA,
        pltpu.SemaphoreType.DMA,
        pltpu.SemaphoreType.DMA,
    ],
)


def _hs_body(x_ref, w_ref, degp_ref, hs_ref, dinv_ref):
    deg = lax.dot_general(degp_ref[0], jnp.ones((NW, 1), jnp.float32),
                          (((0,), (0,)), ((), ())),
                          preferred_element_type=jnp.float32) + 1.0
    dinv = lax.rsqrt(deg)
    h = jnp.dot(x_ref[...], w_ref[...], preferred_element_type=jnp.float32)
    hs_ref[...] = dinv * h
    dinv_ref[...] = dinv


_hs_call = pl.pallas_call(
    _hs_body,
    grid=(NBLK,),
    in_specs=[
        pl.BlockSpec((BLK, D), lambda i: (i, 0)),
        pl.BlockSpec((D, H), lambda i: (0, 0)),
        pl.BlockSpec((1, NW, BLK), lambda i: (i, 0, 0)),
    ],
    out_specs=[
        pl.BlockSpec((BLK, H), lambda i: (i, 0)),
        pl.BlockSpec((BLK, 1), lambda i: (i, 0)),
    ],
    out_shape=[
        jax.ShapeDtypeStruct((N, H), jnp.float32),
        jax.ShapeDtypeStruct((N, 1), jnp.float32),
    ],
)


def _final_body(aggp_ref, hs_ref, dinv_ref, batch_ref, wlin_ref, out_ref,
                acc, cnt):
    i = pl.program_id(0)

    @pl.when(i == 0)
    def _init():
        acc[...] = jnp.zeros_like(acc)
        cnt[...] = jnp.zeros_like(cnt)

    agg = jnp.sum(aggp_ref[...], axis=0)
    r = jnp.maximum(dinv_ref[...] * (agg + hs_ref[...]), 0.0)
    b_ids = batch_ref[...]
    onehot = (b_ids == lax.broadcasted_iota(jnp.int32, (BLK, B), 1)
              ).astype(jnp.float32)
    acc[...] += lax.dot_general(onehot, r, (((0,), (0,)), ((), ())),
                                preferred_element_type=jnp.float32)
    cnt[...] += lax.dot_general(onehot, jnp.ones((BLK, 1), jnp.float32),
                                (((0,), (0,)), ((), ())),
                                preferred_element_type=jnp.float32)

    @pl.when(i == NBLK - 1)
    def _emit():
        pooled = acc[...] / jnp.maximum(cnt[...], 1.0)
        out_ref[...] = jnp.dot(pooled, wlin_ref[...],
                               preferred_element_type=jnp.float32)


_final_call = pl.pallas_call(
    _final_body,
    grid=(NBLK,),
    in_specs=[
        pl.BlockSpec((NC, BLK, H), lambda i: (0, i, 0)),
        pl.BlockSpec((BLK, H), lambda i: (i, 0)),
        pl.BlockSpec((BLK, 1), lambda i: (i, 0)),
        pl.BlockSpec((BLK, 1), lambda i: (i, 0)),
        pl.BlockSpec((H, C), lambda i: (0, 0)),
    ],
    out_specs=pl.BlockSpec((B, C), lambda i: (0, 0)),
    out_shape=jax.ShapeDtypeStruct((B, C), jnp.float32),
    scratch_shapes=[
        pltpu.VMEM((B, H), jnp.float32),
        pltpu.VMEM((B, 1), jnp.float32),
    ],
)


def kernel(x, edge_index, batch, W_conv, W_lin):
    ei = edge_index.astype(jnp.int32)
    pad = E_PAD - E
    cyc = jnp.arange(pad, dtype=jnp.int32)
    src_p = jnp.concatenate([ei[0], cyc % K])
    dst_p = jnp.concatenate([ei[1], N + cyc % (N_PAD - N)])
    src_2d = src_p.reshape(NW, NPH, NB_PH, 2, K // 2)
    dst_2d = dst_p.reshape(NW, NPH, NB_PH, 2, K // 2)

    deg_parts = _deg_kernel(ei.reshape(2 * E))
    degp_t = deg_parts.reshape(NW, NBLK, BLK).transpose(1, 0, 2)

    hs, dinv = _hs_call(x, W_conv, degp_t)

    zeros_stripe = jnp.zeros((ROWS_PER_SUB, H), jnp.float32)
    agg_parts = _msg_kernel(hs, src_2d, dst_2d, zeros_stripe)

    logits = _final_call(agg_parts, hs, dinv,
                         batch.astype(jnp.int32).reshape(N, 1), W_lin)
    return logits


# speedup vs baseline: 1.0536x; 1.0536x over previous
import jax
import jax.numpy as jnp
from jax import lax
from jax.experimental import pallas as pl
from jax.experimental.pallas import tpu as pltpu
from jax.experimental.pallas import tpu_sc as plsc

N = 10000
E = 320000
D = 128
H = 128
C = 10
B = 128

NC, NS = 2, 16
NW = NC * NS
N_PAD = 10240
BLK = 2000
NBLK = N // BLK
K = 128
NB = 80
NPH = 2
NB_PH = NB // NPH
NPAIR = NB_PH // 2
E_PW = NB * K
E_PAD = NW * E_PW
E_PW_DEG = E // NW
ROWS_PER_SUB = N_PAD // NS

_mesh = plsc.VectorSubcoreMesh(
    core_axis_name="c", subcore_axis_name="s", num_cores=NC, num_subcores=NS)


def _deg_body(ei_hbm, out_hbm, dstv, hist):
    c = lax.axis_index("c")
    s = lax.axis_index("s")
    wid = s * NC + c
    zeros16 = jnp.zeros((16,), jnp.float32)

    def _zero(i, carry):
        hist[pl.ds(i * 16, 16)] = zeros16
        return carry

    lax.fori_loop(0, N // 16, _zero, 0)
    pltpu.sync_copy(ei_hbm.at[pl.ds(E + wid * E_PW_DEG, E_PW_DEG)], dstv)
    ones16 = jnp.ones((16,), jnp.float32)

    def _acc(i, carry):
        idx = dstv[pl.ds(i * 16, 16)]
        plsc.addupdate_scatter(hist, [idx], ones16)
        return carry

    lax.fori_loop(0, E_PW_DEG // 16, _acc, 0)
    pltpu.sync_copy(hist, out_hbm.at[wid])


_deg_kernel = pl.kernel(
    _deg_body,
    out_type=jax.ShapeDtypeStruct((NW, N), jnp.float32),
    mesh=_mesh,
    scratch_types=[
        pltpu.VMEM((E_PW_DEG,), jnp.int32),
        pltpu.VMEM((N,), jnp.float32),
    ],
    compiler_params=pltpu.CompilerParams(needs_layout_passes=False),
)


def _msg_body(hs_hbm, src_hbm, dst_hbm, zero_hbm, out_hbm,
              srcv, dstv, rows_a, rows_b, acc_sh,
              gsem_a, gsem_a2, gsem_b, gsem_b2, ssem_a, ssem_b):
    c = lax.axis_index("c")
    s = lax.axis_index("s")
    wid = s * NC + c
    pltpu.sync_copy(zero_hbm, acc_sh.at[pl.ds(s * ROWS_PER_SUB, ROWS_PER_SUB)])
    plsc.subcore_barrier()

    KH = K // 2

    def _gather(j, rows, sem_lo, sem_hi):
        pltpu.async_copy(hs_hbm.at[srcv.at[j, pl.ds(0, KH)]],
                         rows.at[pl.ds(0, KH)], sem_lo)
        pltpu.async_copy(hs_hbm.at[srcv.at[j, pl.ds(KH, KH)]],
                         rows.at[pl.ds(KH, KH)], sem_hi)

    def _gwait(rows, sem_lo, sem_hi):
        pltpu.make_async_copy(hs_hbm.at[srcv.at[0, pl.ds(0, KH)]],
                              rows.at[pl.ds(0, KH)], sem_lo).wait()
        pltpu.make_async_copy(hs_hbm.at[srcv.at[0, pl.ds(KH, KH)]],
                              rows.at[pl.ds(KH, KH)], sem_hi).wait()

    for ph in range(NPH):
        pltpu.sync_copy(src_hbm.at[wid, ph], srcv)
        pltpu.sync_copy(dst_hbm.at[wid, ph], dstv)
        _gather(0, rows_a, gsem_a, gsem_a2)

        def _pair(p, carry):
            j = 2 * p
            _gwait(rows_a, gsem_a, gsem_a2)

            @pl.when(p > 0)
            def _b_free():
                pltpu.make_async_copy(
                    rows_b, acc_sh.at[dstv.at[0]], ssem_b).wait()

            _gather(j + 1, rows_b, gsem_b, gsem_b2)
            pltpu.async_copy(rows_a, acc_sh.at[dstv.at[j]], ssem_a, add=True)
            _gwait(rows_b, gsem_b, gsem_b2)

            @pl.when(p < NPAIR - 1)
            def _a_free():
                pltpu.make_async_copy(
                    rows_a, acc_sh.at[dstv.at[0]], ssem_a).wait()
                _gather(j + 2, rows_a, gsem_a, gsem_a2)

            pltpu.async_copy(rows_b, acc_sh.at[dstv.at[j + 1]], ssem_b,
                             add=True)
            return carry

        lax.fori_loop(0, NPAIR, _pair, 0)
        pltpu.make_async_copy(rows_a, acc_sh.at[dstv.at[0]], ssem_a).wait()
        pltpu.make_async_copy(rows_b, acc_sh.at[dstv.at[0]], ssem_b).wait()
    plsc.subcore_barrier()
    pltpu.sync_copy(acc_sh.at[pl.ds(s * ROWS_PER_SUB, ROWS_PER_SUB)],
                    out_hbm.at[c, pl.ds(s * ROWS_PER_SUB, ROWS_PER_SUB)])


_msg_kernel = pl.kernel(
    _msg_body,
    out_type=jax.ShapeDtypeStruct((NC, N_PAD, H), jnp.float32),
    mesh=_mesh,
    scratch_types=[
        pltpu.VMEM((NB_PH, K), jnp.int32),
        pltpu.VMEM((NB_PH, K), jnp.int32),
        pltpu.VMEM((K, H), jnp.float32),
        pltpu.VMEM((K, H), jnp.float32),
        pltpu.VMEM_SHARED((N_PAD, H), jnp.float32),
        pltpu.SemaphoreType.DMA,
        pltpu.SemaphoreType.DMA,
        pltpu.SemaphoreType.DMA,
        pltpu.SemaphoreType.DMA,
        pltpu.SemaphoreType.DMA,
        pltpu.SemaphoreType.DMA,
    ],
)


def _hs_body(x_ref, w_ref, degp_ref, hs_ref, dinv_ref):
    deg = lax.dot_general(degp_ref[0], jnp.ones((NW, 1), jnp.float32),
                          (((0,), (0,)), ((), ())),
                          preferred_element_type=jnp.float32) + 1.0
    dinv = lax.rsqrt(deg)
    h = jnp.dot(x_ref[...], w_ref[...], preferred_element_type=jnp.float32)
    hs_ref[...] = dinv * h
    dinv_ref[...] = dinv


_hs_call = pl.pallas_call(
    _hs_body,
    grid=(NBLK,),
    in_specs=[
        pl.BlockSpec((BLK, D), lambda i: (i, 0)),
        pl.BlockSpec((D, H), lambda i: (0, 0)),
        pl.BlockSpec((1, NW, BLK), lambda i: (i, 0, 0)),
    ],
    out_specs=[
        pl.BlockSpec((BLK, H), lambda i: (i, 0)),
        pl.BlockSpec((BLK, 1), lambda i: (i, 0)),
    ],
    out_shape=[
        jax.ShapeDtypeStruct((N, H), jnp.float32),
        jax.ShapeDtypeStruct((N, 1), jnp.float32),
    ],
)


def _final_body(aggp_ref, hs_ref, dinv_ref, batch_ref, wlin_ref, out_ref,
                acc, cnt):
    i = pl.program_id(0)

    @pl.when(i == 0)
    def _init():
        acc[...] = jnp.zeros_like(acc)
        cnt[...] = jnp.zeros_like(cnt)

    agg = jnp.sum(aggp_ref[...], axis=0)
    r = jnp.maximum(dinv_ref[...] * (agg + hs_ref[...]), 0.0)
    b_ids = batch_ref[...]
    onehot = (b_ids == lax.broadcasted_iota(jnp.int32, (BLK, B), 1)
              ).astype(jnp.float32)
    acc[...] += lax.dot_general(onehot, r, (((0,), (0,)), ((), ())),
                                preferred_element_type=jnp.float32)
    cnt[...] += lax.dot_general(onehot, jnp.ones((BLK, 1), jnp.float32),
                                (((0,), (0,)), ((), ())),
                                preferred_element_type=jnp.float32)

    @pl.when(i == NBLK - 1)
    def _emit():
        pooled = acc[...] / jnp.maximum(cnt[...], 1.0)
        out_ref[...] = jnp.dot(pooled, wlin_ref[...],
                               preferred_element_type=jnp.float32)


_final_call = pl.pallas_call(
    _final_body,
    grid=(NBLK,),
    in_specs=[
        pl.BlockSpec((NC, BLK, H), lambda i: (0, i, 0)),
        pl.BlockSpec((BLK, H), lambda i: (i, 0)),
        pl.BlockSpec((BLK, 1), lambda i: (i, 0)),
        pl.BlockSpec((BLK, 1), lambda i: (i, 0)),
        pl.BlockSpec((H, C), lambda i: (0, 0)),
    ],
    out_specs=pl.BlockSpec((B, C), lambda i: (0, 0)),
    out_shape=jax.ShapeDtypeStruct((B, C), jnp.float32),
    scratch_shapes=[
        pltpu.VMEM((B, H), jnp.float32),
        pltpu.VMEM((B, 1), jnp.float32),
    ],
)


def kernel(x, edge_index, batch, W_conv, W_lin):
    ei = edge_index.astype(jnp.int32)
    pad = E_PAD - E
    cyc = jnp.arange(pad, dtype=jnp.int32)
    src_p = jnp.concatenate([ei[0], cyc % K])
    dst_p = jnp.concatenate([ei[1], N + cyc % (N_PAD - N)])
    src_2d = src_p.reshape(NW, NPH, NB_PH, K)
    dst_2d = dst_p.reshape(NW, NPH, NB_PH, K)

    deg_parts = _deg_kernel(ei.reshape(2 * E))
    degp_t = deg_parts.reshape(NW, NBLK, BLK).transpose(1, 0, 2)

    hs, dinv = _hs_call(x, W_conv, degp_t)

    zeros_stripe = jnp.zeros((ROWS_PER_SUB, H), jnp.float32)
    agg_parts = _msg_kernel(hs, src_2d, dst_2d, zeros_stripe)

    logits = _final_call(agg_parts, hs, dinv,
                         batch.astype(jnp.int32).reshape(N, 1), W_lin)
    return logits


# speedup vs baseline: 1.0537x; 1.0000x over previous
import jax
import jax.numpy as jnp
from jax import lax
from jax.experimental import pallas as pl
from jax.experimental.pallas import tpu as pltpu
from jax.experimental.pallas import tpu_sc as plsc

N = 10000
E = 320000
D = 128
H = 128
C = 10
B = 128

NC, NS = 2, 16
NW = NC * NS
N_PAD = 10240
BLK = 2000
NBLK = N // BLK
K = 128
NB = 80
NPH = 2
NB_PH = NB // NPH
NPAIR = NB_PH // 2
E_PW = NB * K
E_PAD = NW * E_PW
E_PW_DEG = E // NW
ROWS_PER_SUB = N_PAD // NS

_mesh = plsc.VectorSubcoreMesh(
    core_axis_name="c", subcore_axis_name="s", num_cores=NC, num_subcores=NS)


def _deg_body(ei_hbm, out_hbm, dstv, hist):
    c = lax.axis_index("c")
    s = lax.axis_index("s")
    wid = s * NC + c
    zeros16 = jnp.zeros((16,), jnp.float32)

    def _zero(i, carry):
        hist[pl.ds(i * 16, 16)] = zeros16
        return carry

    lax.fori_loop(0, N // 16, _zero, 0)
    pltpu.sync_copy(ei_hbm.at[pl.ds(E + wid * E_PW_DEG, E_PW_DEG)], dstv)
    ones16 = jnp.ones((16,), jnp.float32)

    def _acc(i, carry):
        idx = dstv[pl.ds(i * 16, 16)]
        plsc.addupdate_scatter(hist, [idx], ones16)
        return carry

    lax.fori_loop(0, E_PW_DEG // 16, _acc, 0)
    pltpu.sync_copy(hist, out_hbm.at[wid])


_deg_kernel = pl.kernel(
    _deg_body,
    out_type=jax.ShapeDtypeStruct((NW, N), jnp.float32),
    mesh=_mesh,
    scratch_types=[
        pltpu.VMEM((E_PW_DEG,), jnp.int32),
        pltpu.VMEM((N,), jnp.float32),
    ],
    compiler_params=pltpu.CompilerParams(needs_layout_passes=False),
)


def _msg_body(hs_hbm, src_hbm, dst_hbm, zero_hbm, out_hbm,
              srcv, dstv, rows_a, rows_b, acc_sh,
              gsem_a, gsem_a2, gsem_b, gsem_b2, ssem_a, ssem_b):
    c = lax.axis_index("c")
    s = lax.axis_index("s")
    wid = s * NC + c
    pltpu.sync_copy(zero_hbm, acc_sh.at[pl.ds(s * ROWS_PER_SUB, ROWS_PER_SUB)])
    plsc.subcore_barrier()

    KH = K // 2

    KQ = K // 4

    def _gather(j, rows, sem_lo, sem_hi):
        for q, sem in ((0, sem_lo), (1, sem_lo), (2, sem_hi), (3, sem_hi)):
            pltpu.async_copy(hs_hbm.at[srcv.at[j, pl.ds(q * KQ, KQ)]],
                             rows.at[pl.ds(q * KQ, KQ)], sem)

    def _gwait(rows, sem_lo, sem_hi):
        for q, sem in ((0, sem_lo), (1, sem_lo), (2, sem_hi), (3, sem_hi)):
            pltpu.make_async_copy(hs_hbm.at[srcv.at[0, pl.ds(q * KQ, KQ)]],
                                  rows.at[pl.ds(q * KQ, KQ)], sem).wait()

    for ph in range(NPH):
        pltpu.sync_copy(src_hbm.at[wid, ph], srcv)
        pltpu.sync_copy(dst_hbm.at[wid, ph], dstv)
        _gather(0, rows_a, gsem_a, gsem_a2)

        def _pair(p, carry):
            j = 2 * p
            _gwait(rows_a, gsem_a, gsem_a2)

            @pl.when(p > 0)
            def _b_free():
                pltpu.make_async_copy(
                    rows_b, acc_sh.at[dstv.at[0]], ssem_b).wait()

            _gather(j + 1, rows_b, gsem_b, gsem_b2)
            pltpu.async_copy(rows_a, acc_sh.at[dstv.at[j]], ssem_a, add=True)
            _gwait(rows_b, gsem_b, gsem_b2)

            @pl.when(p < NPAIR - 1)
            def _a_free():
                pltpu.make_async_copy(
                    rows_a, acc_sh.at[dstv.at[0]], ssem_a).wait()
                _gather(j + 2, rows_a, gsem_a, gsem_a2)

            pltpu.async_copy(rows_b, acc_sh.at[dstv.at[j + 1]], ssem_b,
                             add=True)
            return carry

        lax.fori_loop(0, NPAIR, _pair, 0)
        pltpu.make_async_copy(rows_a, acc_sh.at[dstv.at[0]], ssem_a).wait()
        pltpu.make_async_copy(rows_b, acc_sh.at[dstv.at[0]], ssem_b).wait()
    plsc.subcore_barrier()
    pltpu.sync_copy(acc_sh.at[pl.ds(s * ROWS_PER_SUB, ROWS_PER_SUB)],
                    out_hbm.at[c, pl.ds(s * ROWS_PER_SUB, ROWS_PER_SUB)])


_msg_kernel = pl.kernel(
    _msg_body,
    out_type=jax.ShapeDtypeStruct((NC, N_PAD, H), jnp.float32),
    mesh=_mesh,
    scratch_types=[
        pltpu.VMEM((NB_PH, K), jnp.int32),
        pltpu.VMEM((NB_PH, K), jnp.int32),
        pltpu.VMEM((K, H), jnp.float32),
        pltpu.VMEM((K, H), jnp.float32),
        pltpu.VMEM_SHARED((N_PAD, H), jnp.float32),
        pltpu.SemaphoreType.DMA,
        pltpu.SemaphoreType.DMA,
        pltpu.SemaphoreType.DMA,
        pltpu.SemaphoreType.DMA,
        pltpu.SemaphoreType.DMA,
        pltpu.SemaphoreType.DMA,
    ],
)


def _hs_body(x_ref, w_ref, degp_ref, hs_ref, dinv_ref):
    deg = lax.dot_general(degp_ref[0], jnp.ones((NW, 1), jnp.float32),
                          (((0,), (0,)), ((), ())),
                          preferred_element_type=jnp.float32) + 1.0
    dinv = lax.rsqrt(deg)
    h = jnp.dot(x_ref[...], w_ref[...], preferred_element_type=jnp.float32)
    hs_ref[...] = dinv * h
    dinv_ref[...] = dinv


_hs_call = pl.pallas_call(
    _hs_body,
    grid=(NBLK,),
    in_specs=[
        pl.BlockSpec((BLK, D), lambda i: (i, 0)),
        pl.BlockSpec((D, H), lambda i: (0, 0)),
        pl.BlockSpec((1, NW, BLK), lambda i: (i, 0, 0)),
    ],
    out_specs=[
        pl.BlockSpec((BLK, H), lambda i: (i, 0)),
        pl.BlockSpec((BLK, 1), lambda i: (i, 0)),
    ],
    out_shape=[
        jax.ShapeDtypeStruct((N, H), jnp.float32),
        jax.ShapeDtypeStruct((N, 1), jnp.float32),
    ],
)


def _final_body(aggp_ref, hs_ref, dinv_ref, batch_ref, wlin_ref, out_ref,
                acc, cnt):
    i = pl.program_id(0)

    @pl.when(i == 0)
    def _init():
        acc[...] = jnp.zeros_like(acc)
        cnt[...] = jnp.zeros_like(cnt)

    agg = jnp.sum(aggp_ref[...], axis=0)
    r = jnp.maximum(dinv_ref[...] * (agg + hs_ref[...]), 0.0)
    b_ids = batch_ref[...]
    onehot = (b_ids == lax.broadcasted_iota(jnp.int32, (BLK, B), 1)
              ).astype(jnp.float32)
    acc[...] += lax.dot_general(onehot, r, (((0,), (0,)), ((), ())),
                                preferred_element_type=jnp.float32)
    cnt[...] += lax.dot_general(onehot, jnp.ones((BLK, 1), jnp.float32),
                                (((0,), (0,)), ((), ())),
                                preferred_element_type=jnp.float32)

    @pl.when(i == NBLK - 1)
    def _emit():
        pooled = acc[...] / jnp.maximum(cnt[...], 1.0)
        out_ref[...] = jnp.dot(pooled, wlin_ref[...],
                               preferred_element_type=jnp.float32)


_final_call = pl.pallas_call(
    _final_body,
    grid=(NBLK,),
    in_specs=[
        pl.BlockSpec((NC, BLK, H), lambda i: (0, i, 0)),
        pl.BlockSpec((BLK, H), lambda i: (i, 0)),
        pl.BlockSpec((BLK, 1), lambda i: (i, 0)),
        pl.BlockSpec((BLK, 1), lambda i: (i, 0)),
        pl.BlockSpec((H, C), lambda i: (0, 0)),
    ],
    out_specs=pl.BlockSpec((B, C), lambda i: (0, 0)),
    out_shape=jax.ShapeDtypeStruct((B, C), jnp.float32),
    scratch_shapes=[
        pltpu.VMEM((B, H), jnp.float32),
        pltpu.VMEM((B, 1), jnp.float32),
    ],
)


def kernel(x, edge_index, batch, W_conv, W_lin):
    ei = edge_index.astype(jnp.int32)
    pad = E_PAD - E
    cyc = jnp.arange(pad, dtype=jnp.int32)
    src_p = jnp.concatenate([ei[0], cyc % K])
    dst_p = jnp.concatenate([ei[1], N + cyc % (N_PAD - N)])
    src_2d = src_p.reshape(NW, NPH, NB_PH, K)
    dst_2d = dst_p.reshape(NW, NPH, NB_PH, K)

    deg_parts = _deg_kernel(ei.reshape(2 * E))
    degp_t = deg_parts.reshape(NW, NBLK, BLK).transpose(1, 0, 2)

    hs, dinv = _hs_call(x, W_conv, degp_t)

    zeros_stripe = jnp.zeros((ROWS_PER_SUB, H), jnp.float32)
    agg_parts = _msg_kernel(hs, src_2d, dst_2d, zeros_stripe)

    logits = _final_call(agg_parts, hs, dinv,
                         batch.astype(jnp.int32).reshape(N, 1), W_lin)
    return logits


# speedup vs baseline: 1.0541x; 1.0005x over previous
import jax
import jax.numpy as jnp
from jax import lax
from jax.experimental import pallas as pl
from jax.experimental.pallas import tpu as pltpu
from jax.experimental.pallas import tpu_sc as plsc

N = 10000
E = 320000
D = 128
H = 128
C = 10
B = 128

NC, NS = 2, 16
NW = NC * NS
N_PAD = 10240
BLK = 2000
NBLK = N // BLK
K = 128
NB = 80
NPH = 2
NB_PH = NB // NPH
NPAIR = NB_PH // 2
E_PW = NB * K
E_PAD = NW * E_PW
E_PW_DEG = E // NW
ROWS_PER_SUB = N_PAD // NS

_mesh = plsc.VectorSubcoreMesh(
    core_axis_name="c", subcore_axis_name="s", num_cores=NC, num_subcores=NS)


def _deg_body(ei_hbm, out_hbm, dstv, hist):
    c = lax.axis_index("c")
    s = lax.axis_index("s")
    wid = s * NC + c
    zeros16 = jnp.zeros((16,), jnp.float32)

    def _zero(i, carry):
        hist[pl.ds(i * 16, 16)] = zeros16
        return carry

    lax.fori_loop(0, N // 16, _zero, 0)
    pltpu.sync_copy(ei_hbm.at[pl.ds(E + wid * E_PW_DEG, E_PW_DEG)], dstv)
    ones16 = jnp.ones((16,), jnp.float32)

    def _acc(i, carry):
        idx = dstv[pl.ds(i * 16, 16)]
        plsc.addupdate_scatter(hist, [idx], ones16)
        return carry

    lax.fori_loop(0, E_PW_DEG // 16, _acc, 0)
    pltpu.sync_copy(hist, out_hbm.at[wid])


_deg_kernel = pl.kernel(
    _deg_body,
    out_type=jax.ShapeDtypeStruct((NW, N), jnp.float32),
    mesh=_mesh,
    scratch_types=[
        pltpu.VMEM((E_PW_DEG,), jnp.int32),
        pltpu.VMEM((N,), jnp.float32),
    ],
    compiler_params=pltpu.CompilerParams(needs_layout_passes=False),
)


def _msg_body(hs_hbm, src_hbm, dst_hbm, zero_hbm, out_hbm,
              srcv, dstv, rows_a, rows_b, acc_sh,
              gsem_a, gsem_a2, gsem_b, gsem_b2, ssem_a, ssem_b):
    c = lax.axis_index("c")
    s = lax.axis_index("s")
    wid = s * NC + c
    pltpu.sync_copy(zero_hbm, acc_sh.at[pl.ds(s * ROWS_PER_SUB, ROWS_PER_SUB)])
    plsc.subcore_barrier()

    KH = K // 2

    def _gather(j, rows, sem_lo, sem_hi):
        pltpu.async_copy(hs_hbm.at[srcv.at[j, pl.ds(0, KH)]],
                         rows.at[pl.ds(0, KH)], sem_lo)
        pltpu.async_copy(hs_hbm.at[srcv.at[j, pl.ds(KH, KH)]],
                         rows.at[pl.ds(KH, KH)], sem_hi)

    def _gwait(rows, sem_lo, sem_hi):
        pltpu.make_async_copy(hs_hbm.at[srcv.at[0, pl.ds(0, KH)]],
                              rows.at[pl.ds(0, KH)], sem_lo).wait()
        pltpu.make_async_copy(hs_hbm.at[srcv.at[0, pl.ds(KH, KH)]],
                              rows.at[pl.ds(KH, KH)], sem_hi).wait()

    for ph in range(NPH):
        pltpu.sync_copy(src_hbm.at[wid, ph], srcv)
        pltpu.sync_copy(dst_hbm.at[wid, ph], dstv)
        _gather(0, rows_a, gsem_a, gsem_a2)

        def _pair(p, carry):
            j = 2 * p
            _gwait(rows_a, gsem_a, gsem_a2)

            @pl.when(p > 0)
            def _b_free():
                pltpu.make_async_copy(
                    rows_b, acc_sh.at[dstv.at[0]], ssem_b).wait()

            _gather(j + 1, rows_b, gsem_b, gsem_b2)
            pltpu.async_copy(rows_a, acc_sh.at[dstv.at[j]], ssem_a, add=True)
            _gwait(rows_b, gsem_b, gsem_b2)

            @pl.when(p < NPAIR - 1)
            def _a_free():
                pltpu.make_async_copy(
                    rows_a, acc_sh.at[dstv.at[0]], ssem_a).wait()
                _gather(j + 2, rows_a, gsem_a, gsem_a2)

            pltpu.async_copy(rows_b, acc_sh.at[dstv.at[j + 1]], ssem_b,
                             add=True)
            return carry

        lax.fori_loop(0, NPAIR, _pair, 0)
        pltpu.make_async_copy(rows_a, acc_sh.at[dstv.at[0]], ssem_a).wait()
        pltpu.make_async_copy(rows_b, acc_sh.at[dstv.at[0]], ssem_b).wait()
    plsc.subcore_barrier()
    pltpu.sync_copy(acc_sh.at[pl.ds(s * ROWS_PER_SUB, ROWS_PER_SUB)],
                    out_hbm.at[c, pl.ds(s * ROWS_PER_SUB, ROWS_PER_SUB)])


_msg_kernel = pl.kernel(
    _msg_body,
    out_type=jax.ShapeDtypeStruct((NC, N_PAD, H), jnp.float32),
    mesh=_mesh,
    scratch_types=[
        pltpu.VMEM((NB_PH, K), jnp.int32),
        pltpu.VMEM((NB_PH, K), jnp.int32),
        pltpu.VMEM((K, H), jnp.float32),
        pltpu.VMEM((K, H), jnp.float32),
        pltpu.VMEM_SHARED((N_PAD, H), jnp.float32),
        pltpu.SemaphoreType.DMA,
        pltpu.SemaphoreType.DMA,
        pltpu.SemaphoreType.DMA,
        pltpu.SemaphoreType.DMA,
        pltpu.SemaphoreType.DMA,
        pltpu.SemaphoreType.DMA,
    ],
)


def _hs_body(x_ref, w_ref, degp_ref, hs_ref, dinv_ref):
    deg = lax.dot_general(degp_ref[0], jnp.ones((NW, 1), jnp.float32),
                          (((0,), (0,)), ((), ())),
                          preferred_element_type=jnp.float32) + 1.0
    dinv = lax.rsqrt(deg)
    h = jnp.dot(x_ref[...], w_ref[...], preferred_element_type=jnp.float32)
    hs_ref[...] = dinv * h
    dinv_ref[...] = dinv


_hs_call = pl.pallas_call(
    _hs_body,
    grid=(NBLK,),
    in_specs=[
        pl.BlockSpec((BLK, D), lambda i: (i, 0)),
        pl.BlockSpec((D, H), lambda i: (0, 0)),
        pl.BlockSpec((1, NW, BLK), lambda i: (i, 0, 0)),
    ],
    out_specs=[
        pl.BlockSpec((BLK, H), lambda i: (i, 0)),
        pl.BlockSpec((BLK, 1), lambda i: (i, 0)),
    ],
    out_shape=[
        jax.ShapeDtypeStruct((N, H), jnp.float32),
        jax.ShapeDtypeStruct((N, 1), jnp.float32),
    ],
)


def _final_body(aggp_ref, hs_ref, dinv_ref, batch_ref, wlin_ref, out_ref,
                acc, cnt):
    i = pl.program_id(0)

    @pl.when(i == 0)
    def _init():
        acc[...] = jnp.zeros_like(acc)
        cnt[...] = jnp.zeros_like(cnt)

    agg = jnp.sum(aggp_ref[...], axis=0)
    r = jnp.maximum(dinv_ref[...] * (agg + hs_ref[...]), 0.0)
    b_ids = batch_ref[...]
    onehot = (b_ids == lax.broadcasted_iota(jnp.int32, (BLK, B), 1)
              ).astype(jnp.float32)
    acc[...] += lax.dot_general(onehot, r, (((0,), (0,)), ((), ())),
                                preferred_element_type=jnp.float32)
    cnt[...] += lax.dot_general(onehot, jnp.ones((BLK, 1), jnp.float32),
                                (((0,), (0,)), ((), ())),
                                preferred_element_type=jnp.float32)

    @pl.when(i == NBLK - 1)
    def _emit():
        pooled = acc[...] / jnp.maximum(cnt[...], 1.0)
        out_ref[...] = jnp.dot(pooled, wlin_ref[...],
                               preferred_element_type=jnp.float32)


_final_call = pl.pallas_call(
    _final_body,
    grid=(NBLK,),
    in_specs=[
        pl.BlockSpec((NC, BLK, H), lambda i: (0, i, 0)),
        pl.BlockSpec((BLK, H), lambda i: (i, 0)),
        pl.BlockSpec((BLK, 1), lambda i: (i, 0)),
        pl.BlockSpec((BLK, 1), lambda i: (i, 0)),
        pl.BlockSpec((H, C), lambda i: (0, 0)),
    ],
    out_specs=pl.BlockSpec((B, C), lambda i: (0, 0)),
    out_shape=jax.ShapeDtypeStruct((B, C), jnp.float32),
    scratch_shapes=[
        pltpu.VMEM((B, H), jnp.float32),
        pltpu.VMEM((B, 1), jnp.float32),
    ],
)


def kernel(x, edge_index, batch, W_conv, W_lin):
    ei = edge_index.astype(jnp.int32)
    pad = E_PAD - E
    cyc = jnp.arange(pad, dtype=jnp.int32)
    src_p = jnp.concatenate([ei[0], cyc % K])
    dst_p = jnp.concatenate([ei[1], N + cyc % (N_PAD - N)])
    src_2d = src_p.reshape(NW, NPH, NB_PH, K)
    dst_2d = dst_p.reshape(NW, NPH, NB_PH, K)

    deg_parts = _deg_kernel(ei.reshape(2 * E))
    degp_t = deg_parts.reshape(NW, NBLK, BLK).transpose(1, 0, 2)

    hs, dinv = _hs_call(x, W_conv, degp_t)

    zeros_stripe = jnp.zeros((ROWS_PER_SUB, H), jnp.float32)
    agg_parts = _msg_kernel(hs, src_2d, dst_2d, zeros_stripe)

    logits = _final_call(agg_parts, hs, dinv,
                         batch.astype(jnp.int32).reshape(N, 1), W_lin)
    return logits
